# baseline (speedup 1.0000x reference)
import functools

import jax
import jax.numpy as jnp
from jax import lax
from jax.experimental import pallas as pl
from jax.experimental.pallas import tpu as pltpu
from jax.experimental.pallas import tpu_sc as plsc

D = 128
EDGE_BLK = 512
NODE_BLK = 1000
INTERPRET = False

NC = 2
NS = 16
GCH = 128
SCH = 128


def _sc_mesh():
    return plsc.VectorSubcoreMesh(core_axis_name="c", subcore_axis_name="s")


def _sc_gather(a_tab, b_tab, src, dst):
    e_pad = src.shape[0]
    n_ch = e_pad // (NC * NS * GCH)

    @functools.partial(
        pl.kernel,
        out_type=[jax.ShapeDtypeStruct((e_pad, D), jnp.float32),
                  jax.ShapeDtypeStruct((e_pad, D), jnp.float32)],
        mesh=_sc_mesh(),
        scratch_types=[
            pltpu.VMEM((GCH,), jnp.int32),
            pltpu.VMEM((GCH,), jnp.int32),
            pltpu.VMEM((GCH, D), jnp.float32),
            pltpu.VMEM((GCH, D), jnp.float32),
            pltpu.SemaphoreType.DMA,
            pltpu.SemaphoreType.DMA,
        ],
    )
    def k(a_hbm, b_hbm, src_hbm, dst_hbm, ga_hbm, gb_hbm,
          idxs_v, idxd_v, bufa, bufb, sema, semb):
        wid = lax.axis_index("s") * NC + lax.axis_index("c")
        base0 = wid * n_ch * GCH

        def body(j, carry):
            base = base0 + j * GCH
            pltpu.sync_copy(src_hbm.at[pl.ds(base, GCH)], idxs_v)
            pltpu.sync_copy(dst_hbm.at[pl.ds(base, GCH)], idxd_v)
            ca = pltpu.async_copy(a_hbm.at[idxs_v], bufa, sema)
            cb = pltpu.async_copy(b_hbm.at[idxd_v], bufb, semb)
            ca.wait()
            cb.wait()
            pltpu.sync_copy(bufa, ga_hbm.at[pl.ds(base, GCH)])
            pltpu.sync_copy(bufb, gb_hbm.at[pl.ds(base, GCH)])
            return carry

        lax.fori_loop(0, n_ch, body, 0)

    return k(a_tab, b_tab, src, dst)


def _tables_body(xs_ref, xd_ref, wl_ref, wr_ref, b_ref, a_ref, bt_ref):
    a_ref[...] = jnp.dot(xs_ref[...], wl_ref[...],
                         preferred_element_type=jnp.float32)
    bt_ref[...] = jnp.dot(xd_ref[...], wr_ref[...],
                          preferred_element_type=jnp.float32) + b_ref[...]


def _tables(xs, xd, wa, ba):
    n = xs.shape[0]
    wl = wa[:, :D].T
    wr = wa[:, D:].T
    grid = n // NODE_BLK
    return pl.pallas_call(
        _tables_body,
        grid=(grid,),
        in_specs=[
            pl.BlockSpec((NODE_BLK, D), lambda i: (i, 0)),
            pl.BlockSpec((NODE_BLK, D), lambda i: (i, 0)),
            pl.BlockSpec((D, D), lambda i: (0, 0)),
            pl.BlockSpec((D, D), lambda i: (0, 0)),
            pl.BlockSpec((1, D), lambda i: (0, 0)),
        ],
        out_specs=[
            pl.BlockSpec((NODE_BLK, D), lambda i: (i, 0)),
            pl.BlockSpec((NODE_BLK, D), lambda i: (i, 0)),
        ],
        out_shape=[
            jax.ShapeDtypeStruct((n, D), jnp.float32),
            jax.ShapeDtypeStruct((n, D), jnp.float32),
        ],
        interpret=INTERPRET,
    )(xs, xd, wl, wr, ba[None, :])


def _edge_body(n_edges, ga_ref, gb_ref, w2_ref, b2_ref, itau_ref,
               out_ref, w_ref):
    i = pl.program_id(0)
    h = jnp.maximum(ga_ref[...] + gb_ref[...], 0.0)
    m = jnp.dot(h, w2_ref[...], preferred_element_type=jnp.float32) + b2_ref[...]
    nrm = jnp.sqrt(jnp.sum(m * m, axis=1, keepdims=True))
    row = i * EDGE_BLK + jax.lax.broadcasted_iota(jnp.int32, (EDGE_BLK, 1), 0)
    w = jnp.where(row < n_edges, jnp.exp(-nrm * itau_ref[0, 0]), 0.0)
    out_ref[...] = m * w
    col = jax.lax.broadcasted_iota(jnp.int32, (EDGE_BLK, 16), 1)
    w_ref[...] = jnp.where(col == 0, w, 0.0)


def _edge_mlp(ga, gb, wb, bb, inv_tau, n_edges):
    e_pad = ga.shape[0]
    grid = e_pad // EDGE_BLK
    return pl.pallas_call(
        functools.partial(_edge_body, n_edges),
        grid=(grid,),
        in_specs=[
            pl.BlockSpec((EDGE_BLK, D), lambda i: (i, 0)),
            pl.BlockSpec((EDGE_BLK, D), lambda i: (i, 0)),
            pl.BlockSpec((D, D), lambda i: (0, 0)),
            pl.BlockSpec((1, D), lambda i: (0, 0)),
            pl.BlockSpec(memory_space=pltpu.SMEM),
        ],
        out_specs=[
            pl.BlockSpec((EDGE_BLK, D), lambda i: (i, 0)),
            pl.BlockSpec((EDGE_BLK, 16), lambda i: (i, 0)),
        ],
        out_shape=[
            jax.ShapeDtypeStruct((e_pad, D), jnp.float32),
            jax.ShapeDtypeStruct((e_pad, 16), jnp.float32),
        ],
        interpret=INTERPRET,
    )(ga, gb, wb.T, bb[None, :], inv_tau)


RED_NB = 1024
RED_EB = 512


def _reduce_body(dst_ref, num_ref, w_ref, outn_ref, outd_ref):
    i = pl.program_id(0)
    j = pl.program_id(1)

    @pl.when(j == 0)
    def _():
        outn_ref[...] = jnp.zeros_like(outn_ref)
        outd_ref[...] = jnp.zeros_like(outd_ref)

    node = i * RED_NB + jax.lax.broadcasted_iota(jnp.int32, (RED_NB, RED_EB), 0)
    oh = (node == dst_ref[...]).astype(jnp.float32)
    outn_ref[...] += jnp.dot(oh, num_ref[...],
                             preferred_element_type=jnp.float32)
    outd_ref[...] += jnp.dot(oh, w_ref[...],
                             preferred_element_type=jnp.float32)


def _tc_reduce(num, wrow, dst, n_pad):
    e_pad = num.shape[0]
    grid = (n_pad // RED_NB, e_pad // RED_EB)
    return pl.pallas_call(
        _reduce_body,
        grid=grid,
        in_specs=[
            pl.BlockSpec((1, RED_EB), lambda i, j: (0, j)),
            pl.BlockSpec((RED_EB, D), lambda i, j: (j, 0)),
            pl.BlockSpec((RED_EB, 16), lambda i, j: (j, 0)),
        ],
        out_specs=[
            pl.BlockSpec((RED_NB, D), lambda i, j: (i, 0)),
            pl.BlockSpec((RED_NB, 16), lambda i, j: (i, 0)),
        ],
        out_shape=[
            jax.ShapeDtypeStruct((n_pad, D), jnp.float32),
            jax.ShapeDtypeStruct((n_pad, 16), jnp.float32),
        ],
        interpret=INTERPRET,
    )(dst[None, :], num, wrow)


def _finalize_body(accn_ref, accd_ref, old_ref, out_ref):
    num = accn_ref[...]
    den = accd_ref[:, 0][:, None]
    agg = num / jnp.where(den > 0, den, 1.0)
    out_ref[...] = jnp.where(den > 0, agg, old_ref[...])


def _finalize(accn, accd, old, blk):
    n = old.shape[0]
    grid = n // blk
    return pl.pallas_call(
        _finalize_body,
        grid=(grid,),
        in_specs=[
            pl.BlockSpec((blk, D), lambda i: (i, 0)),
            pl.BlockSpec((blk, 16), lambda i: (i, 0)),
            pl.BlockSpec((blk, D), lambda i: (i, 0)),
        ],
        out_specs=pl.BlockSpec((blk, D), lambda i: (i, 0)),
        out_shape=jax.ShapeDtypeStruct((n, D), jnp.float32),
        interpret=INTERPRET,
    )(accn, accd, old)


def _pad_idx(idx, e_pad):
    return jnp.concatenate(
        [idx, jnp.zeros((e_pad - idx.shape[0],), dtype=idx.dtype)])


def _stage(xs, xd, src, dst, wa, ba, wb, bb, inv_tau, old, num_dst):
    e = src.shape[0]
    e_pad = ((e + 8191) // 8192) * 8192
    src_p = _pad_idx(src, e_pad)
    dst_p = _pad_idx(dst, e_pad)
    a_tab, b_tab = _tables(xs, xd, wa, ba)
    ga, gb = _sc_gather(a_tab, b_tab, src_p, dst_p)
    num, wrow = _edge_mlp(ga, gb, wb, bb, inv_tau, e)
    n_pad = ((num_dst + 2047) // 2048) * 2048
    accn, accd = _tc_reduce(num, wrow, dst_p, n_pad)
    old_pad = jnp.pad(old, ((0, n_pad - num_dst), (0, 0)))
    return _finalize(accn, accd, old_pad, 1024)[:num_dst]


def kernel(tile_feat, rr_feat, edge_t2t, edge_rr2t, edge_t2rr, temperature,
           W1a, b1a, W1b, b1b, W2a, b2a, W2b, b2b, W3a, b3a, W3b, b3b):
    inv_tau = (1.0 / temperature).reshape(1, 1).astype(jnp.float32)
    n_tile = tile_feat.shape[0]
    n_rr = rr_feat.shape[0]
    tile = _stage(tile_feat, tile_feat, edge_t2t[0], edge_t2t[1],
                  W1a, b1a, W1b, b1b, inv_tau, tile_feat, n_tile)
    tile = _stage(rr_feat, tile, edge_rr2t[0], edge_rr2t[1],
                  W2a, b2a, W2b, b2b, inv_tau, tile, n_tile)
    rr = _stage(tile, rr_feat, edge_t2rr[0], edge_t2rr[1],
                W3a, b3a, W3b, b3b, inv_tau, rr_feat, n_rr)
    return tile, rr


# speedup vs baseline: 1.0200x; 1.0200x over previous
import functools

import jax
import jax.numpy as jnp
from jax import lax
from jax.experimental import pallas as pl
from jax.experimental.pallas import tpu as pltpu
from jax.experimental.pallas import tpu_sc as plsc

D = 128
EDGE_BLK = 512
NODE_BLK = 1000
INTERPRET = False

NC = 2
NS = 16
GCH = 128
SCH = 128


def _sc_mesh():
    return plsc.VectorSubcoreMesh(core_axis_name="c", subcore_axis_name="s")


def _sc_gather(a_tab, b_tab, src, dst):
    e_pad = src.shape[0]
    n_ch = e_pad // (NC * NS * GCH)

    @functools.partial(
        pl.kernel,
        out_type=[jax.ShapeDtypeStruct((e_pad, D), jnp.float32),
                  jax.ShapeDtypeStruct((e_pad, D), jnp.float32)],
        mesh=_sc_mesh(),
        scratch_types=[
            pltpu.VMEM((GCH,), jnp.int32),
            pltpu.VMEM((GCH,), jnp.int32),
            pltpu.VMEM((GCH, D), jnp.float32),
            pltpu.VMEM((GCH, D), jnp.float32),
            pltpu.SemaphoreType.DMA,
            pltpu.SemaphoreType.DMA,
        ],
    )
    def k(a_hbm, b_hbm, src_hbm, dst_hbm, ga_hbm, gb_hbm,
          idxs_v, idxd_v, bufa, bufb, sema, semb):
        wid = lax.axis_index("s") * NC + lax.axis_index("c")
        base0 = wid * n_ch * GCH

        def body(j, carry):
            base = base0 + j * GCH
            pltpu.sync_copy(src_hbm.at[pl.ds(base, GCH)], idxs_v)
            pltpu.sync_copy(dst_hbm.at[pl.ds(base, GCH)], idxd_v)
            ca = pltpu.async_copy(a_hbm.at[idxs_v], bufa, sema)
            cb = pltpu.async_copy(b_hbm.at[idxd_v], bufb, semb)
            ca.wait()
            cb.wait()
            pltpu.sync_copy(bufa, ga_hbm.at[pl.ds(base, GCH)])
            pltpu.sync_copy(bufb, gb_hbm.at[pl.ds(base, GCH)])
            return carry

        lax.fori_loop(0, n_ch, body, 0)

    return k(a_tab, b_tab, src, dst)


def _tables_body(xs_ref, xd_ref, wl_ref, wr_ref, b_ref, a_ref, bt_ref):
    a_ref[...] = jnp.dot(xs_ref[...], wl_ref[...],
                         preferred_element_type=jnp.float32)
    bt_ref[...] = jnp.dot(xd_ref[...], wr_ref[...],
                          preferred_element_type=jnp.float32) + b_ref[...]


def _tables(xs, xd, wa, ba):
    n = xs.shape[0]
    wl = wa[:, :D].T
    wr = wa[:, D:].T
    grid = n // NODE_BLK
    return pl.pallas_call(
        _tables_body,
        grid=(grid,),
        in_specs=[
            pl.BlockSpec((NODE_BLK, D), lambda i: (i, 0)),
            pl.BlockSpec((NODE_BLK, D), lambda i: (i, 0)),
            pl.BlockSpec((D, D), lambda i: (0, 0)),
            pl.BlockSpec((D, D), lambda i: (0, 0)),
            pl.BlockSpec((1, D), lambda i: (0, 0)),
        ],
        out_specs=[
            pl.BlockSpec((NODE_BLK, D), lambda i: (i, 0)),
            pl.BlockSpec((NODE_BLK, D), lambda i: (i, 0)),
        ],
        out_shape=[
            jax.ShapeDtypeStruct((n, D), jnp.float32),
            jax.ShapeDtypeStruct((n, D), jnp.float32),
        ],
        interpret=INTERPRET,
    )(xs, xd, wl, wr, ba[None, :])


def _edge_body(n_edges, ga_ref, gb_ref, w2_ref, b2_ref, itau_ref,
               out_ref, w_ref):
    i = pl.program_id(0)
    h = jnp.maximum(ga_ref[...] + gb_ref[...], 0.0)
    m = jnp.dot(h, w2_ref[...], preferred_element_type=jnp.float32) + b2_ref[...]
    nrm = jnp.sqrt(jnp.sum(m * m, axis=1, keepdims=True))
    row = i * EDGE_BLK + jax.lax.broadcasted_iota(jnp.int32, (EDGE_BLK, 1), 0)
    w = jnp.where(row < n_edges, jnp.exp(-nrm * itau_ref[0, 0]), 0.0)
    out_ref[...] = m * w
    col = jax.lax.broadcasted_iota(jnp.int32, (EDGE_BLK, 16), 1)
    w_ref[...] = jnp.where(col == 0, w, 0.0)


def _edge_mlp(ga, gb, wb, bb, inv_tau, n_edges):
    e_pad = ga.shape[0]
    grid = e_pad // EDGE_BLK
    return pl.pallas_call(
        functools.partial(_edge_body, n_edges),
        grid=(grid,),
        in_specs=[
            pl.BlockSpec((EDGE_BLK, D), lambda i: (i, 0)),
            pl.BlockSpec((EDGE_BLK, D), lambda i: (i, 0)),
            pl.BlockSpec((D, D), lambda i: (0, 0)),
            pl.BlockSpec((1, D), lambda i: (0, 0)),
            pl.BlockSpec(memory_space=pltpu.SMEM),
        ],
        out_specs=[
            pl.BlockSpec((EDGE_BLK, D), lambda i: (i, 0)),
            pl.BlockSpec((EDGE_BLK, 16), lambda i: (i, 0)),
        ],
        out_shape=[
            jax.ShapeDtypeStruct((e_pad, D), jnp.float32),
            jax.ShapeDtypeStruct((e_pad, 16), jnp.float32),
        ],
        interpret=INTERPRET,
    )(ga, gb, wb.T, bb[None, :], inv_tau)


RED_NB = 1024
RED_EB = 512


def _reduce_body(dst_ref, num_ref, w_ref, outn_ref, outd_ref):
    i = pl.program_id(0)
    j = pl.program_id(1)

    @pl.when(j == 0)
    def _():
        outn_ref[...] = jnp.zeros_like(outn_ref)
        outd_ref[...] = jnp.zeros_like(outd_ref)

    node = i * RED_NB + jax.lax.broadcasted_iota(jnp.int32, (RED_NB, RED_EB), 0)
    oh = (node == dst_ref[...]).astype(jnp.bfloat16)
    outn_ref[...] += jnp.dot(oh, num_ref[...],
                             preferred_element_type=jnp.float32)
    outd_ref[...] += jnp.dot(oh, w_ref[...],
                             preferred_element_type=jnp.float32)


def _tc_reduce(num, wrow, dst, n_pad):
    e_pad = num.shape[0]
    grid = (n_pad // RED_NB, e_pad // RED_EB)
    return pl.pallas_call(
        _reduce_body,
        grid=grid,
        in_specs=[
            pl.BlockSpec((1, RED_EB), lambda i, j: (0, j)),
            pl.BlockSpec((RED_EB, D), lambda i, j: (j, 0)),
            pl.BlockSpec((RED_EB, 16), lambda i, j: (j, 0)),
        ],
        out_specs=[
            pl.BlockSpec((RED_NB, D), lambda i, j: (i, 0)),
            pl.BlockSpec((RED_NB, 16), lambda i, j: (i, 0)),
        ],
        out_shape=[
            jax.ShapeDtypeStruct((n_pad, D), jnp.float32),
            jax.ShapeDtypeStruct((n_pad, 16), jnp.float32),
        ],
        interpret=INTERPRET,
    )(dst[None, :], num.astype(jnp.bfloat16), wrow.astype(jnp.bfloat16))


def _finalize_body(accn_ref, accd_ref, old_ref, out_ref):
    num = accn_ref[...]
    den = accd_ref[:, 0][:, None]
    agg = num / jnp.where(den > 0, den, 1.0)
    out_ref[...] = jnp.where(den > 0, agg, old_ref[...])


def _finalize(accn, accd, old, blk):
    n = old.shape[0]
    grid = n // blk
    return pl.pallas_call(
        _finalize_body,
        grid=(grid,),
        in_specs=[
            pl.BlockSpec((blk, D), lambda i: (i, 0)),
            pl.BlockSpec((blk, 16), lambda i: (i, 0)),
            pl.BlockSpec((blk, D), lambda i: (i, 0)),
        ],
        out_specs=pl.BlockSpec((blk, D), lambda i: (i, 0)),
        out_shape=jax.ShapeDtypeStruct((n, D), jnp.float32),
        interpret=INTERPRET,
    )(accn, accd, old)


def _pad_idx(idx, e_pad):
    return jnp.concatenate(
        [idx, jnp.zeros((e_pad - idx.shape[0],), dtype=idx.dtype)])


def _stage(xs, xd, src, dst, wa, ba, wb, bb, inv_tau, old, num_dst):
    e = src.shape[0]
    e_pad = ((e + 8191) // 8192) * 8192
    src_p = _pad_idx(src, e_pad)
    dst_p = _pad_idx(dst, e_pad)
    a_tab, b_tab = _tables(xs, xd, wa, ba)
    ga, gb = _sc_gather(a_tab, b_tab, src_p, dst_p)
    num, wrow = _edge_mlp(ga, gb, wb, bb, inv_tau, e)
    n_pad = ((num_dst + 2047) // 2048) * 2048
    accn, accd = _tc_reduce(num, wrow, dst_p, n_pad)
    old_pad = jnp.pad(old, ((0, n_pad - num_dst), (0, 0)))
    return _finalize(accn, accd, old_pad, 1024)[:num_dst]


def kernel(tile_feat, rr_feat, edge_t2t, edge_rr2t, edge_t2rr, temperature,
           W1a, b1a, W1b, b1b, W2a, b2a, W2b, b2b, W3a, b3a, W3b, b3b):
    inv_tau = (1.0 / temperature).reshape(1, 1).astype(jnp.float32)
    n_tile = tile_feat.shape[0]
    n_rr = rr_feat.shape[0]
    tile = _stage(tile_feat, tile_feat, edge_t2t[0], edge_t2t[1],
                  W1a, b1a, W1b, b1b, inv_tau, tile_feat, n_tile)
    tile = _stage(rr_feat, tile, edge_rr2t[0], edge_rr2t[1],
                  W2a, b2a, W2b, b2b, inv_tau, tile, n_tile)
    rr = _stage(tile, rr_feat, edge_t2rr[0], edge_t2rr[1],
                W3a, b3a, W3b, b3b, inv_tau, rr_feat, n_rr)
    return tile, rr
